# baseline (speedup 1.0000x reference)
import jax
import jax.numpy as jnp
import numpy as np
from jax.experimental import pallas as pl
from jax.experimental.pallas import tpu as pltpu

_EMB = 48


def _fourier_basis() -> np.ndarray:
    e = (2.0 ** np.arange(_EMB // 6)).astype(np.float32) * np.pi
    z = np.zeros(_EMB // 6, dtype=np.float32)
    return np.stack([
        np.concatenate([e, z, z]),
        np.concatenate([z, e, z]),
        np.concatenate([z, z, e]),
    ])


def _body(idx_ref, feat_ref, table_ref, w_ref, b_ref, out_ref, acc_ref):
    T, D = out_ref.shape
    pos = jax.lax.dot_general(
        feat_ref[...], w_ref[...], (((1,), (0,)), ((), ())),
        preferred_element_type=jnp.float32)
    pos = pos + b_ref[...]

    U = 16

    def tok(g, carry):
        base = g * U
        for u in range(U):
            t = base + u
            r0 = table_ref[pl.ds(idx_ref[t, 0], 8), :]
            r1 = table_ref[pl.ds(idx_ref[t, 1], 8), :]
            r2 = table_ref[pl.ds(idx_ref[t, 2], 8), :]
            r3 = table_ref[pl.ds(idx_ref[t, 3], 8), :]
            acc_ref[t] = (r0 + r1) + (r2 + r3)
        return carry

    jax.lax.fori_loop(0, T // U, tok, 0)
    out_ref[...] = acc_ref[...].reshape(T, D) + pos


def kernel(input, centers, table, W, b):
    B, K, L = input.shape
    V, D = table.shape
    S = D // 128
    N = B * L
    T = next(t for t in (512, 256, 128, 64, 32, 16, 8, 4, 2, 1) if N % t == 0)

    basis = jnp.asarray(_fourier_basis())
    projections = jnp.einsum('bnd,de->bne', centers, basis)
    pos_emb = jnp.concatenate(
        [jnp.sin(projections), jnp.cos(projections)], axis=2)
    pos_in = jnp.concatenate([centers, pos_emb], axis=2)
    feat = pos_in.reshape(N, _EMB + 3)

    idx = (input.transpose(1, 0, 2).reshape(K, N).T * S).astype(jnp.int32)
    table2 = table.reshape(V * S, 128)
    b2 = b.reshape(1, D)

    out2 = pl.pallas_call(
        _body,
        grid=(N // T,),
        in_specs=[
            pl.BlockSpec((T, K), lambda i: (i, 0), memory_space=pltpu.SMEM),
            pl.BlockSpec((T, _EMB + 3), lambda i: (i, 0)),
            pl.BlockSpec((V * S, 128), lambda i: (0, 0)),
            pl.BlockSpec(W.shape, lambda i: (0, 0)),
            pl.BlockSpec((1, D), lambda i: (0, 0)),
        ],
        out_specs=pl.BlockSpec((T, D), lambda i: (i, 0)),
        out_shape=jax.ShapeDtypeStruct((N, D), jnp.float32),
        scratch_shapes=[pltpu.VMEM((T, S, 128), jnp.float32)],
        compiler_params=pltpu.CompilerParams(
            dimension_semantics=("parallel",)),
    )(idx, feat, table2, W, b2)
    return out2.reshape(B, L, D)


# speedup vs baseline: 1.1197x; 1.1197x over previous
import jax
import jax.numpy as jnp
import numpy as np
from jax.experimental import pallas as pl
from jax.experimental.pallas import tpu as pltpu

_EMB = 48


def _fourier_basis() -> np.ndarray:
    e = (2.0 ** np.arange(_EMB // 6)).astype(np.float32) * np.pi
    z = np.zeros(_EMB // 6, dtype=np.float32)
    return np.stack([
        np.concatenate([e, z, z]),
        np.concatenate([z, e, z]),
        np.concatenate([z, z, e]),
    ])


def _body(idx_ref, feat_ref, table_ref, w_ref, b_ref, out_ref, acc_ref):
    T, D = out_ref.shape
    pos = jax.lax.dot_general(
        feat_ref[...], w_ref[...], (((1,), (0,)), ((), ())),
        preferred_element_type=jnp.float32)
    pos = pos + b_ref[...]

    U = 16

    def tok(g, carry):
        base = g * U
        for u in range(U):
            t = base + u
            acc_ref[t] = ((table_ref[idx_ref[0, t]] + table_ref[idx_ref[1, t]])
                          + (table_ref[idx_ref[2, t]] + table_ref[idx_ref[3, t]]))
        return carry

    jax.lax.fori_loop(0, T // U, tok, 0)
    out_ref[...] = acc_ref[...].reshape(T, D) + pos


def kernel(input, centers, table, W, b):
    B, K, L = input.shape
    V, D = table.shape
    S = D // 128
    N = B * L
    T = next(t for t in (512, 256, 128, 64, 32, 16, 8, 4, 2, 1) if N % t == 0)

    basis = jnp.asarray(_fourier_basis())
    projections = jnp.einsum('bnd,de->bne', centers, basis)
    pos_emb = jnp.concatenate(
        [jnp.sin(projections), jnp.cos(projections)], axis=2)
    pos_in = jnp.concatenate([centers, pos_emb], axis=2)
    feat = pos_in.reshape(N, _EMB + 3)

    idx = input.transpose(1, 0, 2).reshape(K, N)
    table3 = table.reshape(V, S, 128)
    b2 = b.reshape(1, D)

    out2 = pl.pallas_call(
        _body,
        grid=(N // T,),
        in_specs=[
            pl.BlockSpec((K, T), lambda i: (0, i), memory_space=pltpu.SMEM),
            pl.BlockSpec((T, _EMB + 3), lambda i: (i, 0)),
            pl.BlockSpec((V, S, 128), lambda i: (0, 0, 0)),
            pl.BlockSpec(W.shape, lambda i: (0, 0)),
            pl.BlockSpec((1, D), lambda i: (0, 0)),
        ],
        out_specs=pl.BlockSpec((T, D), lambda i: (i, 0)),
        out_shape=jax.ShapeDtypeStruct((N, D), jnp.float32),
        scratch_shapes=[pltpu.VMEM((T, S, 128), jnp.float32)],
        compiler_params=pltpu.CompilerParams(
            dimension_semantics=("parallel",)),
    )(idx, feat, table3, W, b2)
    return out2.reshape(B, L, D)


# speedup vs baseline: 1.1412x; 1.0192x over previous
import jax
import jax.numpy as jnp
import numpy as np
from jax.experimental import pallas as pl
from jax.experimental.pallas import tpu as pltpu

_EMB = 48


def _fourier_basis() -> np.ndarray:
    e = (2.0 ** np.arange(_EMB // 6)).astype(np.float32) * np.pi
    z = np.zeros(_EMB // 6, dtype=np.float32)
    return np.stack([
        np.concatenate([e, z, z]),
        np.concatenate([z, e, z]),
        np.concatenate([z, z, e]),
    ])


def _body(idx_ref, feat_ref, table_ref, w_ref, b_ref, out_ref, acc_ref):
    T, D = out_ref.shape
    pos = jax.lax.dot_general(
        feat_ref[...], w_ref[...], (((1,), (0,)), ((), ())),
        preferred_element_type=jnp.float32)
    pos = pos + b_ref[...]

    U = 32

    def tok(g, carry):
        base = g * U
        for u in range(U):
            t = base + u
            acc_ref[t] = ((table_ref[idx_ref[0, t]] + table_ref[idx_ref[1, t]])
                          + (table_ref[idx_ref[2, t]] + table_ref[idx_ref[3, t]]))
        return carry

    jax.lax.fori_loop(0, T // U, tok, 0)
    out_ref[...] = acc_ref[...].reshape(T, D) + pos


def kernel(input, centers, table, W, b):
    B, K, L = input.shape
    V, D = table.shape
    S = D // 128
    N = B * L
    T = next(t for t in (512, 256, 128, 64, 32, 16, 8, 4, 2, 1) if N % t == 0)

    basis = jnp.asarray(_fourier_basis())
    projections = jnp.einsum('bnd,de->bne', centers, basis)
    pos_emb = jnp.concatenate(
        [jnp.sin(projections), jnp.cos(projections)], axis=2)
    pos_in = jnp.concatenate([centers, pos_emb], axis=2)
    feat = pos_in.reshape(N, _EMB + 3)

    idx = input.transpose(1, 0, 2).reshape(K, N)
    table3 = table.reshape(V, S, 128)
    b2 = b.reshape(1, D)

    out2 = pl.pallas_call(
        _body,
        grid=(N // T,),
        in_specs=[
            pl.BlockSpec((K, T), lambda i: (0, i), memory_space=pltpu.SMEM),
            pl.BlockSpec((T, _EMB + 3), lambda i: (i, 0)),
            pl.BlockSpec((V, S, 128), lambda i: (0, 0, 0)),
            pl.BlockSpec(W.shape, lambda i: (0, 0)),
            pl.BlockSpec((1, D), lambda i: (0, 0)),
        ],
        out_specs=pl.BlockSpec((T, D), lambda i: (i, 0)),
        out_shape=jax.ShapeDtypeStruct((N, D), jnp.float32),
        scratch_shapes=[pltpu.VMEM((T, S, 128), jnp.float32)],
        compiler_params=pltpu.CompilerParams(
            dimension_semantics=("parallel",)),
    )(idx, feat, table3, W, b2)
    return out2.reshape(B, L, D)


# speedup vs baseline: 1.1715x; 1.0265x over previous
import jax
import jax.numpy as jnp
import numpy as np
from jax.experimental import pallas as pl
from jax.experimental.pallas import tpu as pltpu

_EMB = 48


def _fourier_basis() -> np.ndarray:
    e = (2.0 ** np.arange(_EMB // 6)).astype(np.float32) * np.pi
    z = np.zeros(_EMB // 6, dtype=np.float32)
    return np.stack([
        np.concatenate([e, z, z]),
        np.concatenate([z, e, z]),
        np.concatenate([z, z, e]),
    ])


def _body(idx_ref, feat_ref, table_ref, w_ref, b_ref, out_ref, acc_ref):
    T, D = out_ref.shape
    pos = jax.lax.dot_general(
        feat_ref[...], w_ref[...], (((1,), (0,)), ((), ())),
        preferred_element_type=jnp.float32)
    pos = pos + b_ref[...]

    U = 32

    def tok(g, carry):
        base = g * U
        for u in range(U):
            t = base + u
            acc_ref[t] = ((table_ref[idx_ref[0, t]] + table_ref[idx_ref[1, t]])
                          + (table_ref[idx_ref[2, t]] + table_ref[idx_ref[3, t]]))
        return carry

    jax.lax.fori_loop(0, T // U, tok, 0)
    out_ref[...] = acc_ref[...].reshape(T, D) + pos


def kernel(input, centers, table, W, b):
    B, K, L = input.shape
    V, D = table.shape
    S = D // 128
    N = B * L
    T = next(t for t in (1024, 512, 256, 128, 64, 32, 16, 8, 4, 2, 1)
             if N % t == 0)

    basis = jnp.asarray(_fourier_basis())
    projections = jnp.einsum('bnd,de->bne', centers, basis)
    pos_emb = jnp.concatenate(
        [jnp.sin(projections), jnp.cos(projections)], axis=2)
    pos_in = jnp.concatenate([centers, pos_emb], axis=2)
    feat = pos_in.reshape(N, _EMB + 3)

    idx = input.transpose(1, 0, 2).reshape(K, N)
    table3 = table.reshape(V, S, 128)
    b2 = b.reshape(1, D)

    out2 = pl.pallas_call(
        _body,
        grid=(N // T,),
        in_specs=[
            pl.BlockSpec((K, T), lambda i: (0, i), memory_space=pltpu.SMEM),
            pl.BlockSpec((T, _EMB + 3), lambda i: (i, 0)),
            pl.BlockSpec((V, S, 128), lambda i: (0, 0, 0)),
            pl.BlockSpec(W.shape, lambda i: (0, 0)),
            pl.BlockSpec((1, D), lambda i: (0, 0)),
        ],
        out_specs=pl.BlockSpec((T, D), lambda i: (i, 0)),
        out_shape=jax.ShapeDtypeStruct((N, D), jnp.float32),
        scratch_shapes=[pltpu.VMEM((T, S, 128), jnp.float32)],
        compiler_params=pltpu.CompilerParams(
            dimension_semantics=("parallel",)),
    )(idx, feat, table3, W, b2)
    return out2.reshape(B, L, D)
